# Initial kernel scaffold; baseline (speedup 1.0000x reference)
import jax
import jax.numpy as jnp
from jax.experimental import pallas as pl


def kernel(tier_indices, scope_indices, tier_table, scope_table, W, b):
    raise NotImplementedError("write your pallas kernel here")


# speedup vs baseline: 4.4017x; 4.4017x over previous
import functools

import jax
import jax.numpy as jnp
from jax import lax
from jax.experimental import pallas as pl
from jax.experimental.pallas import tpu as pltpu
from jax.experimental.pallas import tpu_sc as plsc


def _combo_body(tt_ref, st_ref, w_ref, b_ref, out_ref):
    h = tt_ref.shape[1]
    n_t = tt_ref.shape[0]
    n_s = st_ref.shape[0]
    rows = out_ref.shape[0]
    tp = jnp.dot(tt_ref[...], w_ref[0:h, :], preferred_element_type=jnp.float32)
    sp = jnp.dot(st_ref[...], w_ref[h:2 * h, :], preferred_element_type=jnp.float32)
    stacked = jnp.concatenate([tp, sp], axis=0)
    i = lax.broadcasted_iota(jnp.int32, (rows, n_t + n_s), 0)
    j = lax.broadcasted_iota(jnp.int32, (rows, n_t + n_s), 1)
    tsel = (j < n_t) & (i // n_s == j)
    ssel = (j >= n_t) & (i % n_s == j - n_t)
    sel = jnp.where(tsel | ssel, 1.0, 0.0).astype(jnp.float32)
    out_ref[...] = (
        jnp.dot(sel, stacked, preferred_element_type=jnp.float32) + b_ref[...]
    )


def _build_combo(tier_table, scope_table, W, b, rows):
    h = tier_table.shape[1]
    return pl.pallas_call(
        _combo_body,
        out_shape=jax.ShapeDtypeStruct((rows, h), jnp.float32),
    )(tier_table, scope_table, W, b.reshape(1, h))


def _sc_lookup(tier_flat, scope_flat, combo, n_scope):
    n = tier_flat.shape[0]
    rows, h = combo.shape
    combo_flat = combo.reshape(rows * h)
    info = plsc.get_sparse_core_info()
    nc, ns, nl = info.num_cores, info.num_subcores, info.num_lanes
    nw = nc * ns
    ch = 1024
    r = n // nw
    g = r // ch
    assert r * nw == n and g * ch == r

    mesh = plsc.VectorSubcoreMesh(core_axis_name="c", subcore_axis_name="s")

    @functools.partial(
        pl.kernel,
        out_type=jax.ShapeDtypeStruct((n * h,), jnp.float32),
        mesh=mesh,
        compiler_params=pltpu.CompilerParams(needs_layout_passes=False),
        scratch_types=[
            pltpu.VMEM((rows * h,), jnp.float32),
            pltpu.VMEM((ch,), jnp.int32),
            pltpu.VMEM((ch,), jnp.int32),
            pltpu.VMEM((ch * h,), jnp.float32),
        ],
    )
    def body(tier_hbm, scope_hbm, combo_hbm, out_hbm,
             combo_v, tier_v, scope_v, rows_v):
        wid = lax.axis_index("s") * nc + lax.axis_index("c")
        base = wid * r
        pltpu.sync_copy(combo_hbm, combo_v)
        lanes32 = lax.iota(jnp.int32, nl) * h

        def chunk(gi, carry):
            off = base + gi * ch
            pltpu.sync_copy(tier_hbm.at[pl.ds(off, ch)], tier_v)
            pltpu.sync_copy(scope_hbm.at[pl.ds(off, ch)], scope_v)

            def group(i, c2):
                t = tier_v[pl.ds(i * nl, nl)]
                s = scope_v[pl.ds(i * nl, nl)]
                cvec = (t * n_scope + s) * h
                dvec = lanes32 + i * (nl * h)
                for c in range(h):
                    vals = plsc.load_gather(combo_v, [cvec + c])
                    plsc.store_scatter(rows_v, [dvec + c], vals)
                return c2

            lax.fori_loop(0, ch // nl, group, 0)
            pltpu.sync_copy(rows_v, out_hbm.at[pl.ds(off * h, ch * h)])
            return carry

        lax.fori_loop(0, g, chunk, 0)

    return body(tier_flat, scope_flat, combo_flat)


def kernel(tier_indices, scope_indices, tier_table, scope_table, W, b):
    bsz, seq = tier_indices.shape
    n_tier, h = tier_table.shape
    n_scope = scope_table.shape[0]
    n = bsz * seq
    rows = ((n_tier * n_scope + 7) // 8) * 8
    combo = _build_combo(tier_table, scope_table, W, b, rows)
    tier_flat = tier_indices.reshape(n).astype(jnp.int32)
    scope_flat = scope_indices.reshape(n).astype(jnp.int32)
    out = _sc_lookup(tier_flat, scope_flat, combo, n_scope)
    return out.reshape(bsz, seq, h)


# speedup vs baseline: 5.3910x; 1.2248x over previous
import functools

import jax
import jax.numpy as jnp
from jax import lax
from jax.experimental import pallas as pl
from jax.experimental.pallas import tpu as pltpu
from jax.experimental.pallas import tpu_sc as plsc


def _combo_body(tt_ref, st_ref, w_ref, b_ref, out_ref):
    h = tt_ref.shape[1]
    n_t = tt_ref.shape[0]
    n_s = st_ref.shape[0]
    rows = out_ref.shape[0]
    tp = jnp.dot(tt_ref[...], w_ref[0:h, :], preferred_element_type=jnp.float32)
    sp = jnp.dot(st_ref[...], w_ref[h:2 * h, :], preferred_element_type=jnp.float32)
    stacked = jnp.concatenate([tp, sp], axis=0)
    i = lax.broadcasted_iota(jnp.int32, (rows, n_t + n_s), 0)
    j = lax.broadcasted_iota(jnp.int32, (rows, n_t + n_s), 1)
    tsel = (j < n_t) & (i // n_s == j)
    ssel = (j >= n_t) & (i % n_s == j - n_t)
    sel = jnp.where(tsel | ssel, 1.0, 0.0).astype(jnp.float32)
    out_ref[...] = (
        jnp.dot(sel, stacked, preferred_element_type=jnp.float32) + b_ref[...]
    )


def _build_combo(tier_table, scope_table, W, b, rows):
    h = tier_table.shape[1]
    return pl.pallas_call(
        _combo_body,
        out_shape=jax.ShapeDtypeStruct((rows, h), jnp.float32),
    )(tier_table, scope_table, W, b.reshape(1, h))


def _sc_lookup(tier_flat, scope_flat, combo, n_scope):
    n = tier_flat.shape[0]
    rows, h = combo.shape
    combo_flat = combo.reshape(rows * h)
    info = plsc.get_sparse_core_info()
    nc, ns, nl = info.num_cores, info.num_subcores, info.num_lanes
    nw = nc * ns
    ch = 1024
    r = n // nw
    g = r // ch
    assert r * nw == n and g * ch == r

    mesh = plsc.VectorSubcoreMesh(core_axis_name="c", subcore_axis_name="s")

    @functools.partial(
        pl.kernel,
        out_type=jax.ShapeDtypeStruct((n * h,), jnp.float32),
        mesh=mesh,
        compiler_params=pltpu.CompilerParams(needs_layout_passes=False),
        scratch_types=[
            pltpu.VMEM((rows * h,), jnp.float32),
            pltpu.VMEM((ch,), jnp.int32),
            pltpu.VMEM((ch,), jnp.int32),
            pltpu.VMEM((ch * h,), jnp.float32),
        ],
    )
    def body(tier_hbm, scope_hbm, combo_hbm, out_hbm,
             combo_v, tier_v, scope_v, rows_v):
        wid = lax.axis_index("s") * nc + lax.axis_index("c")
        base = wid * r
        pltpu.sync_copy(combo_hbm, combo_v)
        lanes32 = lax.iota(jnp.int32, nl) * h

        def chunk(gi, carry):
            off = base + gi * ch
            pltpu.sync_copy(tier_hbm.at[pl.ds(off, ch)], tier_v)
            pltpu.sync_copy(scope_hbm.at[pl.ds(off, ch)], scope_v)

            def group(i, c2):
                t = tier_v[pl.ds(i * nl, nl)]
                s = scope_v[pl.ds(i * nl, nl)]
                cvec = (t * n_scope + s) * h
                dvec = lanes32 + i * (nl * h)
                vals = [plsc.load_gather(combo_v, [cvec + c]) for c in range(h)]
                for c in range(h):
                    plsc.store_scatter(rows_v, [dvec + c], vals[c])
                return c2

            lax.fori_loop(0, ch // nl, group, 0)
            pltpu.sync_copy(rows_v, out_hbm.at[pl.ds(off * h, ch * h)])
            return carry

        lax.fori_loop(0, g, chunk, 0)

    return body(tier_flat, scope_flat, combo_flat)


def kernel(tier_indices, scope_indices, tier_table, scope_table, W, b):
    bsz, seq = tier_indices.shape
    n_tier, h = tier_table.shape
    n_scope = scope_table.shape[0]
    n = bsz * seq
    rows = ((n_tier * n_scope + 7) // 8) * 8
    combo = _build_combo(tier_table, scope_table, W, b, rows)
    tier_flat = tier_indices.reshape(n).astype(jnp.int32)
    scope_flat = scope_indices.reshape(n).astype(jnp.int32)
    out = _sc_lookup(tier_flat, scope_flat, combo, n_scope)
    return out.reshape(bsz, seq, h)


# speedup vs baseline: 5.7511x; 1.0668x over previous
import functools

import jax
import jax.numpy as jnp
from jax import lax
from jax.experimental import pallas as pl
from jax.experimental.pallas import tpu as pltpu
from jax.experimental.pallas import tpu_sc as plsc


def _combo_body(tt_ref, st_ref, w_ref, b_ref, out_ref):
    h = tt_ref.shape[1]
    n_t = tt_ref.shape[0]
    n_s = st_ref.shape[0]
    rows = out_ref.shape[0]
    tp = jnp.dot(tt_ref[...], w_ref[0:h, :], preferred_element_type=jnp.float32)
    sp = jnp.dot(st_ref[...], w_ref[h:2 * h, :], preferred_element_type=jnp.float32)
    stacked = jnp.concatenate([tp, sp], axis=0)
    i = lax.broadcasted_iota(jnp.int32, (rows, n_t + n_s), 0)
    j = lax.broadcasted_iota(jnp.int32, (rows, n_t + n_s), 1)
    tsel = (j < n_t) & (i // n_s == j)
    ssel = (j >= n_t) & (i % n_s == j - n_t)
    sel = jnp.where(tsel | ssel, 1.0, 0.0).astype(jnp.float32)
    out_ref[...] = (
        jnp.dot(sel, stacked, preferred_element_type=jnp.float32) + b_ref[...]
    )


def _build_combo(tier_table, scope_table, W, b, rows):
    h = tier_table.shape[1]
    return pl.pallas_call(
        _combo_body,
        out_shape=jax.ShapeDtypeStruct((rows, h), jnp.float32),
    )(tier_table, scope_table, W, b.reshape(1, h))


def _sc_lookup(tier_flat, scope_flat, combo, n_scope):
    n = tier_flat.shape[0]
    rows, h = combo.shape
    combo_flat = combo.reshape(rows * h)
    info = plsc.get_sparse_core_info()
    nc, ns, nl = info.num_cores, info.num_subcores, info.num_lanes
    nw = nc * ns
    ch = 1024
    r = n // nw
    g = r // ch
    assert r * nw == n and g * ch == r

    mesh = plsc.VectorSubcoreMesh(core_axis_name="c", subcore_axis_name="s")

    @functools.partial(
        pl.kernel,
        out_type=jax.ShapeDtypeStruct((n * h,), jnp.float32),
        mesh=mesh,
        compiler_params=pltpu.CompilerParams(needs_layout_passes=False),
        scratch_types=[
            pltpu.VMEM((rows * h,), jnp.float32),
            pltpu.VMEM((ch,), jnp.int32),
            pltpu.VMEM((ch,), jnp.int32),
            pltpu.VMEM((ch,), jnp.int32),
            pltpu.VMEM((ch,), jnp.int32),
            pltpu.VMEM((ch * h,), jnp.float32),
            pltpu.VMEM((ch * h,), jnp.float32),
            pltpu.SemaphoreType.DMA,
            pltpu.SemaphoreType.DMA,
            pltpu.SemaphoreType.DMA,
            pltpu.SemaphoreType.DMA,
            pltpu.SemaphoreType.DMA,
            pltpu.SemaphoreType.DMA,
        ],
    )
    def body(tier_hbm, scope_hbm, combo_hbm, out_hbm,
             combo_v, tier_v0, tier_v1, scope_v0, scope_v1, rows_v0, rows_v1,
             st0, st1, ss0, ss1, so0, so1):
        wid = lax.axis_index("s") * nc + lax.axis_index("c")
        base = wid * r
        pltpu.sync_copy(combo_hbm, combo_v)
        lanes32 = lax.iota(jnp.int32, nl) * h
        tiers = (tier_v0, tier_v1)
        scopes = (scope_v0, scope_v1)
        rowsb = (rows_v0, rows_v1)
        sts = (st0, st1)
        sss = (ss0, ss1)
        sos = (so0, so1)

        def issue_in(gi, p):
            off = base + gi * ch
            pltpu.async_copy(tier_hbm.at[pl.ds(off, ch)], tiers[p], sts[p])
            pltpu.async_copy(scope_hbm.at[pl.ds(off, ch)], scopes[p], sss[p])

        def compute(p):
            def group(i, c2):
                t = tiers[p][pl.ds(i * nl, nl)]
                s = scopes[p][pl.ds(i * nl, nl)]
                cvec = (t * n_scope + s) * h
                dvec = lanes32 + i * (nl * h)
                vals = [plsc.load_gather(combo_v, [cvec + c]) for c in range(h)]
                for c in range(h):
                    plsc.store_scatter(rowsb[p], [dvec + c], vals[c])
                return c2

            lax.fori_loop(0, ch // nl, group, 0)

        issue_in(0, 0)
        npairs = g // 2

        def pair(k, carry):
            for p in (0, 1):
                gi = k * 2 + p
                off = base + gi * ch
                pltpu.make_async_copy(
                    tier_hbm.at[pl.ds(0, ch)], tiers[p], sts[p]).wait()
                pltpu.make_async_copy(
                    scope_hbm.at[pl.ds(0, ch)], scopes[p], sss[p]).wait()

                @pl.when(gi + 1 < g)
                def _():
                    issue_in(gi + 1, 1 - p)

                @pl.when(k >= 1)
                def _():
                    pltpu.make_async_copy(
                        rowsb[p], out_hbm.at[pl.ds(0, ch * h)], sos[p]).wait()

                compute(p)
                pltpu.async_copy(
                    rowsb[p], out_hbm.at[pl.ds(off * h, ch * h)], sos[p])
            return carry

        lax.fori_loop(0, npairs, pair, 0)
        for p in (0, 1):
            pltpu.make_async_copy(
                rowsb[p], out_hbm.at[pl.ds(0, ch * h)], sos[p]).wait()

    return body(tier_flat, scope_flat, combo_flat)


def kernel(tier_indices, scope_indices, tier_table, scope_table, W, b):
    bsz, seq = tier_indices.shape
    n_tier, h = tier_table.shape
    n_scope = scope_table.shape[0]
    n = bsz * seq
    rows = ((n_tier * n_scope + 7) // 8) * 8
    combo = _build_combo(tier_table, scope_table, W, b, rows)
    tier_flat = tier_indices.reshape(n).astype(jnp.int32)
    scope_flat = scope_indices.reshape(n).astype(jnp.int32)
    out = _sc_lookup(tier_flat, scope_flat, combo, n_scope)
    return out.reshape(bsz, seq, h)


# speedup vs baseline: 11.2856x; 1.9623x over previous
import functools

import jax
import jax.numpy as jnp
from jax import lax
from jax.experimental import pallas as pl
from jax.experimental.pallas import tpu as pltpu
from jax.experimental.pallas import tpu_sc as plsc


def _combo_body(tt_ref, st_ref, w_ref, b_ref, out_ref):
    h = tt_ref.shape[1]
    n_t = tt_ref.shape[0]
    n_s = st_ref.shape[0]
    rows = out_ref.shape[0]
    tp = jnp.dot(tt_ref[...], w_ref[0:h, :], preferred_element_type=jnp.float32)
    sp = jnp.dot(st_ref[...], w_ref[h:2 * h, :], preferred_element_type=jnp.float32)
    stacked = jnp.concatenate([tp, sp], axis=0)
    i = lax.broadcasted_iota(jnp.int32, (rows, n_t + n_s), 0)
    j = lax.broadcasted_iota(jnp.int32, (rows, n_t + n_s), 1)
    tsel = (j < n_t) & (i // n_s == j)
    ssel = (j >= n_t) & (i % n_s == j - n_t)
    sel = jnp.where(tsel | ssel, 1.0, 0.0).astype(jnp.float32)
    out_ref[...] = (
        jnp.dot(sel, stacked, preferred_element_type=jnp.float32) + b_ref[...]
    )


def _build_combo(tier_table, scope_table, W, b, rows):
    h = tier_table.shape[1]
    return pl.pallas_call(
        _combo_body,
        out_shape=jax.ShapeDtypeStruct((rows, h), jnp.float32),
    )(tier_table, scope_table, W, b.reshape(1, h))


def _sc_lookup(tier_flat, scope_flat, combo, n_scope):
    n = tier_flat.shape[0]
    rows, h = combo.shape
    combo_flat = combo.reshape(rows * h)
    info = plsc.get_sparse_core_info()
    nc, ns, nl = info.num_cores, info.num_subcores, info.num_lanes
    nw = nc * ns
    ch = 1024
    r = n // nw
    g = r // ch
    assert r * nw == n and g * ch == r

    mesh = plsc.VectorSubcoreMesh(core_axis_name="c", subcore_axis_name="s")

    @functools.partial(
        pl.kernel,
        out_type=jax.ShapeDtypeStruct((n * h,), jnp.float32),
        mesh=mesh,
        compiler_params=pltpu.CompilerParams(needs_layout_passes=False),
        scratch_types=[
            pltpu.VMEM((rows * h,), jnp.float32),
            pltpu.VMEM((ch,), jnp.int32),
            pltpu.VMEM((ch,), jnp.int32),
            pltpu.VMEM((ch,), jnp.int32),
            pltpu.VMEM((ch,), jnp.int32),
            pltpu.VMEM((ch * h,), jnp.float32),
            pltpu.VMEM((ch * h,), jnp.float32),
            pltpu.SemaphoreType.DMA,
            pltpu.SemaphoreType.DMA,
            pltpu.SemaphoreType.DMA,
            pltpu.SemaphoreType.DMA,
            pltpu.SemaphoreType.DMA,
            pltpu.SemaphoreType.DMA,
        ],
    )
    def body(tier_hbm, scope_hbm, combo_hbm, out_hbm,
             combo_v, tier_v0, tier_v1, scope_v0, scope_v1, rows_v0, rows_v1,
             st0, st1, ss0, ss1, so0, so1):
        wid = lax.axis_index("s") * nc + lax.axis_index("c")
        base = wid * r
        pltpu.sync_copy(combo_hbm, combo_v)
        lanes32 = lax.iota(jnp.int32, nl) * h
        tiers = (tier_v0, tier_v1)
        scopes = (scope_v0, scope_v1)
        rowsb = (rows_v0, rows_v1)
        sts = (st0, st1)
        sss = (ss0, ss1)
        sos = (so0, so1)

        def issue_in(gi, p):
            off = base + gi * ch
            pltpu.async_copy(tier_hbm.at[pl.ds(off, ch)], tiers[p], sts[p])
            pltpu.async_copy(scope_hbm.at[pl.ds(off, ch)], scopes[p], sss[p])

        iota = lax.iota(jnp.int32, nl)

        def compute(p):
            def group(i, c2):
                t = tiers[p][pl.ds(i * nl, nl)]
                s = scopes[p][pl.ds(i * nl, nl)]
                cvec = (t * n_scope + s) * h
                for j in range(nl):
                    bj = lax.gather(
                        cvec,
                        jnp.full((nl, 1), j, jnp.int32),
                        lax.GatherDimensionNumbers(
                            offset_dims=(),
                            collapsed_slice_dims=(0,),
                            start_index_map=(0,),
                        ),
                        (1,),
                        mode=lax.GatherScatterMode.PROMISE_IN_BOUNDS,
                    )
                    halves = [
                        plsc.load_gather(combo_v, [bj + iota + q * nl])
                        for q in range(h // nl)
                    ]
                    tok_off = (i * nl + j) * h
                    for q in range(h // nl):
                        rowsb[p][pl.ds(tok_off + q * nl, nl)] = halves[q]
                return c2

            lax.fori_loop(0, ch // nl, group, 0)

        issue_in(0, 0)
        npairs = g // 2

        def pair(k, carry):
            for p in (0, 1):
                gi = k * 2 + p
                off = base + gi * ch
                pltpu.make_async_copy(
                    tier_hbm.at[pl.ds(0, ch)], tiers[p], sts[p]).wait()
                pltpu.make_async_copy(
                    scope_hbm.at[pl.ds(0, ch)], scopes[p], sss[p]).wait()

                @pl.when(gi + 1 < g)
                def _():
                    issue_in(gi + 1, 1 - p)

                @pl.when(k >= 1)
                def _():
                    pltpu.make_async_copy(
                        rowsb[p], out_hbm.at[pl.ds(0, ch * h)], sos[p]).wait()

                compute(p)
                pltpu.async_copy(
                    rowsb[p], out_hbm.at[pl.ds(off * h, ch * h)], sos[p])
            return carry

        lax.fori_loop(0, npairs, pair, 0)
        for p in (0, 1):
            pltpu.make_async_copy(
                rowsb[p], out_hbm.at[pl.ds(0, ch * h)], sos[p]).wait()

    return body(tier_flat, scope_flat, combo_flat)


def kernel(tier_indices, scope_indices, tier_table, scope_table, W, b):
    bsz, seq = tier_indices.shape
    n_tier, h = tier_table.shape
    n_scope = scope_table.shape[0]
    n = bsz * seq
    rows = ((n_tier * n_scope + 7) // 8) * 8
    combo = _build_combo(tier_table, scope_table, W, b, rows)
    tier_flat = tier_indices.reshape(n).astype(jnp.int32)
    scope_flat = scope_indices.reshape(n).astype(jnp.int32)
    out = _sc_lookup(tier_flat, scope_flat, combo, n_scope)
    return out.reshape(bsz, seq, h)
